# Initial kernel scaffold; baseline (speedup 1.0000x reference)
import jax
import jax.numpy as jnp
from jax.experimental import pallas as pl


def kernel(positions, numbers, neighbor_matrix, covalent_radii, r4r2, c6_reference, coord_num_ref):
    raise NotImplementedError("write your pallas kernel here")


# speedup vs baseline: 1.2326x; 1.2326x over previous
import jax
import jax.numpy as jnp
from jax.experimental import pallas as pl

BOHR_TO_ANGSTROM = 0.529177210544
ANGSTROM_TO_BOHR = 1.0 / BOHR_TO_ANGSTROM
HARTREE_TO_EV = 27.211386245981
A1 = 0.3981
A2 = 4.4211
S8 = 1.9889
S6 = 1.0
K1 = 16.0
K3 = -4.0


def _wu_kernel(cn_ref, cnref_ref, w_ref, u_ref):
    dc = cn_ref[...] - cnref_ref[...]
    w = jnp.exp(K3 * dc * dc)
    w_ref[...] = w
    u_ref[...] = 2.0 * K3 * dc * w


def kernel(positions, numbers, neighbor_matrix, covalent_radii, r4r2, c6_reference, coord_num_ref):
    n = positions.shape[0]
    self_idx = jnp.arange(n, dtype=neighbor_matrix.dtype)[:, None]
    valid = (neighbor_matrix < n) & (neighbor_matrix != self_idx)
    j = jnp.where(valid, neighbor_matrix, 0)
    p = positions * ANGSTROM_TO_BOHR
    z = numbers
    zj = z[j]
    vm = valid.astype(jnp.float32)

    rij = p[j] - p[:, None, :]
    d2 = jnp.sum(rij * rij, axis=-1)
    d2m = jnp.where(valid, d2, 1.0)
    r = jnp.sqrt(d2m)
    rc = covalent_radii[z][:, None] + covalent_radii[zj]
    sig = 1.0 / (1.0 + jnp.exp(-K1 * (rc / r - 1.0)))
    cn = jnp.sum(vm * sig, axis=1)

    cnref_g = coord_num_ref[z]
    npad = 10240
    cn_p = jnp.zeros((npad, 5), jnp.float32).at[:n].set(cn[:, None])
    cnref_p = jnp.zeros((npad, 5), jnp.float32).at[:n].set(cnref_g)
    w_p, u_p = pl.pallas_call(
        _wu_kernel,
        out_shape=(jax.ShapeDtypeStruct((npad, 5), jnp.float32),
                   jax.ShapeDtypeStruct((npad, 5), jnp.float32)),
        grid=(8,),
        in_specs=[pl.BlockSpec((npad // 8, 5), lambda i: (i, 0)),
                  pl.BlockSpec((npad // 8, 5), lambda i: (i, 0))],
        out_specs=(pl.BlockSpec((npad // 8, 5), lambda i: (i, 0)),
                   pl.BlockSpec((npad // 8, 5), lambda i: (i, 0))),
    )(cn_p, cnref_p)
    w = w_p[:n]
    u = u_p[:n]
    ws = jnp.sum(w, axis=1)
    us = jnp.sum(u, axis=1)

    M = c6_reference[z[:, None], zj]
    wjv = w[j]
    Mwj = jnp.einsum('naqr,nar->naq', M, wjv)
    num = jnp.einsum('nq,naq->na', w, Mwj)
    dni = jnp.einsum('nq,naq->na', u, Mwj)
    Muj = jnp.einsum('naqr,nar->naq', M, u[j])
    dnj = jnp.einsum('nq,naq->na', w, Muj)
    den = jnp.maximum(ws[:, None] * ws[j], 1e-30)
    c6 = num / den
    dc6_dcni = (dni - c6 * us[:, None] * ws[j]) / den
    dc6_dcnj = (dnj - c6 * ws[:, None] * us[j]) / den

    qq = 3.0 * r4r2[z][:, None] * r4r2[zj]
    r0 = jnp.sqrt(qq)
    fd = A1 * r0 + A2
    fd2 = fd * fd
    f6 = fd2 * fd2 * fd2
    f8 = f6 * fd2
    r2 = r * r
    r4 = r2 * r2
    r6 = r4 * r2
    r8 = r4 * r4
    inv6 = 1.0 / (r6 + f6)
    inv8 = 1.0 / (r8 + f8)
    gfact = S6 * inv6 + S8 * qq * inv8
    energy = 0.5 * jnp.sum(vm * (-c6) * gfact)

    A = -0.5 * vm * gfact
    dEdcn = jnp.sum(A * dc6_dcni, axis=1)
    dEdcn = dEdcn + jnp.zeros(n).at[j].add(A * dc6_dcnj)

    depair_dr = c6 * (6.0 * S6 * r4 * r * inv6 * inv6 + 8.0 * S8 * qq * r6 * r * inv8 * inv8)
    dsig_dr = -sig * (1.0 - sig) * K1 * rc / r2
    g_r = vm * (0.5 * depair_dr + dEdcn[:, None] * dsig_dr)
    gvec = (g_r / r)[:, :, None] * rij
    grad = -jnp.sum(gvec, axis=1)
    grad = grad + jnp.zeros((n, 3)).at[j].add(gvec)
    forces = -grad * (HARTREE_TO_EV / BOHR_TO_ANGSTROM)
    energy = energy * HARTREE_TO_EV
    return (jnp.reshape(energy, (1,)), forces, cn)


# speedup vs baseline: 37.6126x; 30.5146x over previous
import functools

import jax
import jax.numpy as jnp
from jax import lax
from jax.experimental import pallas as pl
from jax.experimental.pallas import tpu as pltpu
from jax.experimental.pallas import tpu_sc as plsc

BOHR_TO_ANGSTROM = 0.529177210544
ANGSTROM_TO_BOHR = 1.0 / BOHR_TO_ANGSTROM
HARTREE_TO_EV = 27.211386245981
A1 = 0.3981
A2 = 4.4211
S8 = 1.9889
S6 = 1.0
K1 = 16.0
K3 = -4.0

N = 10000
NBR = 32
NPAD = 10240
NW = 32
BLK = NPAD // NW
G = 8
CH = BLK // G
MROW = 32
L = 16

_MESH = plsc.VectorSubcoreMesh(core_axis_name="c", subcore_axis_name="s")


def _rsqrt_nr(x):
    i = plsc.bitcast(x, jnp.int32)
    i = jnp.int32(0x5F3759DF) - (i >> 1)
    y = plsc.bitcast(i, jnp.float32)
    for _ in range(3):
        y = y * (1.5 - 0.5 * x * y * y)
    return y


def _splat_f(x):
    return jnp.broadcast_to(x, (L,)).astype(jnp.float32)


def _splat_i(x):
    return jnp.broadcast_to(x, (L,)).astype(jnp.int32)


def _pass_a(px, py, pz, z_t, cov_t, nbr, cn_out, tq_out,
            px_v, py_v, pz_v, z_v, cov_v, nbr_v, tq_v, cn_v):
    core = lax.axis_index("c")
    sub = lax.axis_index("s")
    wid = core * 16 + sub
    base = wid * BLK
    iota = lax.iota(jnp.int32, L)
    lane0 = iota == 0

    pltpu.sync_copy(px, px_v)
    pltpu.sync_copy(py, py_v)
    pltpu.sync_copy(pz, pz_v)
    pltpu.sync_copy(z_t, z_v)
    pltpu.sync_copy(cov_t, cov_v)

    def chunk_body(c, _):
        row0 = base + c * G
        pltpu.sync_copy(nbr.at[pl.ds(row0, G)], nbr_v)

        def atom_body(a, _):
            ag = row0 + a
            agv = _splat_i(ag)
            ziv = plsc.load_gather(z_v, [agv])
            covi = plsc.load_gather(cov_v, [ziv])
            xi = plsc.load_gather(px_v, [agv])
            yi = plsc.load_gather(py_v, [agv])
            zi = plsc.load_gather(pz_v, [agv])
            cn_a = jnp.float32(0.0)
            for h in range(2):
                nb = nbr_v[a, pl.ds(h * L, L)]
                valid = (nb < N) & (nb != agv)
                jj = jnp.where(valid, nb, 0)
                vm = jnp.where(valid, 1.0, 0.0).astype(jnp.float32)
                dx = plsc.load_gather(px_v, [jj]) - xi
                dy = plsc.load_gather(py_v, [jj]) - yi
                dz = plsc.load_gather(pz_v, [jj]) - zi
                d2 = dx * dx + dy * dy + dz * dz
                d2m = jnp.where(valid, d2, 1.0)
                y = _rsqrt_nr(d2m)
                zjv = plsc.load_gather(z_v, [jj])
                covj = plsc.load_gather(cov_v, [zjv])
                rc = covi + covj
                ee = jnp.exp(-K1 * (rc * y - 1.0))
                sig = 1.0 / (1.0 + ee)
                cn_a = cn_a + jnp.sum(vm * sig)
                tq = vm * sig * (1.0 - sig) * K1 * rc * (y * y * y)
                tq_v[a, pl.ds(h * L, L)] = tq
            plsc.store_scatter(cn_v, [_splat_i(c * G + a)], _splat_f(cn_a),
                               mask=lane0)
            return 0
        lax.fori_loop(0, G, atom_body, 0)
        pltpu.sync_copy(tq_v, tq_out.at[pl.ds(row0, G)])
        return 0
    lax.fori_loop(0, CH, chunk_body, 0)
    pltpu.sync_copy(cn_v, cn_out.at[pl.ds(base, BLK)])


_SC_PARAMS = pltpu.CompilerParams(needs_layout_passes=False, use_tc_tiling_on_sc=False)

_pass_a_call = functools.partial(
    pl.kernel,
    out_type=(jax.ShapeDtypeStruct((NPAD,), jnp.float32),
              jax.ShapeDtypeStruct((NPAD, NBR), jnp.float32)),
    mesh=_MESH,
    compiler_params=_SC_PARAMS,
    scratch_types=[
        pltpu.VMEM((NPAD,), jnp.float32),
        pltpu.VMEM((NPAD,), jnp.float32),
        pltpu.VMEM((NPAD,), jnp.float32),
        pltpu.VMEM((NPAD,), jnp.int32),
        pltpu.VMEM((96,), jnp.float32),
        pltpu.VMEM((G, NBR), jnp.int32),
        pltpu.VMEM((G, NBR), jnp.float32),
        pltpu.VMEM((BLK,), jnp.float32),
    ],
)(_pass_a)


def _pass_b(px, py, pz, z_t, cn_t, r4_t, cnr_t, c6_2d, nbr, zeros1,
            d_out, i_out, e_out, ds_out,
            px_v, py_v, pz_v, z_v, cn_v, r4_v, cnr_v, nbr_v, m_v, midx_v,
            jb_v, zj_v, valj_v, ds_v, w_v, u_v, ws_v, us_v, iv_v, e_v,
            dacc, sem):
    core = lax.axis_index("c")
    sub = lax.axis_index("s")
    wid = core * 16 + sub
    base = wid * BLK
    iota = lax.iota(jnp.int32, L)
    lane0 = iota == 0
    zvec = jnp.zeros((L,), jnp.float32)

    pltpu.sync_copy(px, px_v)
    pltpu.sync_copy(py, py_v)
    pltpu.sync_copy(pz, pz_v)
    pltpu.sync_copy(z_t, z_v)
    pltpu.sync_copy(cn_t, cn_v)
    pltpu.sync_copy(r4_t, r4_v)
    pltpu.sync_copy(cnr_t, cnr_v)

    @pl.when(sub == 0)
    def _():
        pltpu.sync_copy(zeros1, dacc)
    plsc.subcore_barrier()

    def wu_body(k, _):
        lidx = k * L + iota
        al = lidx // 5
        q = lidx - al * 5
        ag = base + al
        cnv = plsc.load_gather(cn_v, [ag])
        zv = plsc.load_gather(z_v, [ag])
        cnr = plsc.load_gather(cnr_v, [zv * 5 + q])
        dcv = cnv - cnr
        wv = jnp.exp(K3 * dcv * dcv)
        plsc.store_scatter(w_v, [lidx], wv)
        plsc.store_scatter(u_v, [lidx], 2.0 * K3 * dcv * wv)
        return 0
    lax.fori_loop(0, BLK * 5 // L, wu_body, 0)

    def ws_body(m, _):
        av = m * L + iota
        aw = zvec
        au = zvec
        for q in range(5):
            aw = aw + plsc.load_gather(w_v, [av * 5 + q])
            au = au + plsc.load_gather(u_v, [av * 5 + q])
        plsc.store_scatter(ws_v, [av], aw)
        plsc.store_scatter(us_v, [av], au)
        return 0
    lax.fori_loop(0, BLK // L, ws_body, 0)

    def chunk_body(c, e_acc):
        row0 = base + c * G
        pltpu.sync_copy(nbr.at[pl.ds(row0, G)], nbr_v)

        def ib(a, _):
            agv = _splat_i(row0 + a)
            ziv = plsc.load_gather(z_v, [agv])
            for h in range(2):
                nb = nbr_v[a, pl.ds(h * L, L)]
                valid = (nb < N) & (nb != agv)
                jj = jnp.where(valid, nb, 0)
                zjv = plsc.load_gather(z_v, [jj])
                pos = a * NBR + h * L + iota
                plsc.store_scatter(midx_v, [pos], ziv * 95 + zjv)
                plsc.store_scatter(jb_v, [pos], jj)
                plsc.store_scatter(zj_v, [pos], zjv)
            return 0
        lax.fori_loop(0, G, ib, 0)
        pltpu.async_copy(c6_2d.at[midx_v], m_v, sem).wait()

        def ab(a, e_acc):
            al = c * G + a
            agv = _splat_i(base + al)
            xi = plsc.load_gather(px_v, [agv])
            yi = plsc.load_gather(py_v, [agv])
            zi3 = plsc.load_gather(pz_v, [agv])
            ziv = plsc.load_gather(z_v, [agv])
            r4i = plsc.load_gather(r4_v, [ziv])
            alv = _splat_i(al)
            wi = [plsc.load_gather(w_v, [alv * 5 + q]) for q in range(5)]
            ui = [plsc.load_gather(u_v, [alv * 5 + q]) for q in range(5)]
            wsi = plsc.load_gather(ws_v, [alv])
            usi = plsc.load_gather(us_v, [alv])
            s_i = jnp.float32(0.0)
            for h in range(2):
                nb = nbr_v[a, pl.ds(h * L, L)]
                valid = (nb < N) & (nb != agv)
                jj = jnp.where(valid, nb, 0)
                vm = jnp.where(valid, 1.0, 0.0).astype(jnp.float32)
                pos = a * NBR + h * L + iota
                zjv = plsc.load_gather(zj_v, [pos])
                dx = plsc.load_gather(px_v, [jj]) - xi
                dy = plsc.load_gather(py_v, [jj]) - yi
                dz = plsc.load_gather(pz_v, [jj]) - zi3
                d2 = dx * dx + dy * dy + dz * dz
                d2m = jnp.where(valid, d2, 1.0)
                cnj = plsc.load_gather(cn_v, [jj])
                r4j = plsc.load_gather(r4_v, [zjv])
                wj = []
                uj = []
                for rj in range(5):
                    cnr = plsc.load_gather(cnr_v, [zjv * 5 + rj])
                    dcj = cnj - cnr
                    wjr = jnp.exp(K3 * dcj * dcj)
                    wj.append(wjr)
                    uj.append(2.0 * K3 * dcj * wjr)
                wsj = wj[0] + wj[1] + wj[2] + wj[3] + wj[4]
                usj = uj[0] + uj[1] + uj[2] + uj[3] + uj[4]
                num = zvec
                dni = zvec
                dnj = zvec
                for q in range(5):
                    for rj in range(5):
                        mv = plsc.load_gather(m_v, [pos, _splat_i(q * 5 + rj)])
                        t1 = mv * wi[q]
                        num = num + t1 * wj[rj]
                        dni = dni + (mv * ui[q]) * wj[rj]
                        dnj = dnj + t1 * uj[rj]
                den_raw = wsi * wsj
                den = jnp.maximum(den_raw, 1e-30)
                m30 = den_raw >= 1e-30
                c6 = num / den
                dc6i = (dni - jnp.where(m30, c6 * usi * wsj, 0.0)) / den
                dc6j = (dnj - jnp.where(m30, c6 * wsi * usj, 0.0)) / den
                qq = 3.0 * r4i * r4j
                yq = _rsqrt_nr(qq)
                r0 = qq * yq
                fd = A1 * r0 + A2
                fd2 = fd * fd
                f6 = fd2 * fd2 * fd2
                f8 = f6 * fd2
                r2 = d2m
                r4p = r2 * r2
                r6 = r4p * r2
                r8 = r4p * r4p
                inv6 = 1.0 / (r6 + f6)
                inv8 = 1.0 / (r8 + f8)
                gfact = S6 * inv6 + S8 * qq * inv8
                af = -0.5 * vm * gfact
                e_acc = e_acc + af * c6
                plsc.store_scatter(valj_v, [pos], af * dc6j)
                ds_v[a, pl.ds(h * L, L)] = vm * 0.5 * c6 * (
                    6.0 * S6 * r4p * inv6 * inv6 + 8.0 * S8 * qq * r6 * inv8 * inv8)
                s_i = s_i + jnp.sum(af * dc6i)
            plsc.store_scatter(iv_v, [alv], _splat_f(s_i), mask=lane0)
            return e_acc
        e_acc = lax.fori_loop(0, G, ab, e_acc)
        pltpu.sync_copy(ds_v, ds_out.at[pl.ds(row0, G)])
        pltpu.sync_copy(valj_v, dacc.at[jb_v], add=True)
        return e_acc

    e_acc = lax.fori_loop(0, CH, chunk_body, jnp.zeros((L,), jnp.float32))
    plsc.subcore_barrier()

    @pl.when(sub == 0)
    def _():
        pltpu.sync_copy(dacc, d_out.at[pl.ds(core * NPAD, NPAD)])
    e_v[...] = e_acc
    pltpu.sync_copy(e_v, e_out.at[pl.ds(wid * L, L)])
    pltpu.sync_copy(iv_v, i_out.at[pl.ds(wid * BLK, BLK)])


_pass_b_call = functools.partial(
    pl.kernel,
    out_type=(jax.ShapeDtypeStruct((2 * NPAD,), jnp.float32),
              jax.ShapeDtypeStruct((NW * BLK,), jnp.float32),
              jax.ShapeDtypeStruct((NW * L,), jnp.float32),
              jax.ShapeDtypeStruct((NPAD, NBR), jnp.float32)),
    mesh=_MESH,
    compiler_params=_SC_PARAMS,
    scratch_types=[
        pltpu.VMEM((NPAD,), jnp.float32),
        pltpu.VMEM((NPAD,), jnp.float32),
        pltpu.VMEM((NPAD,), jnp.float32),
        pltpu.VMEM((NPAD,), jnp.int32),
        pltpu.VMEM((NPAD,), jnp.float32),
        pltpu.VMEM((96,), jnp.float32),
        pltpu.VMEM((480,), jnp.float32),
        pltpu.VMEM((G, NBR), jnp.int32),
        pltpu.VMEM((G * NBR, MROW), jnp.float32),
        pltpu.VMEM((G * NBR,), jnp.int32),
        pltpu.VMEM((G * NBR,), jnp.int32),
        pltpu.VMEM((G * NBR,), jnp.int32),
        pltpu.VMEM((G * NBR,), jnp.float32),
        pltpu.VMEM((G, NBR), jnp.float32),
        pltpu.VMEM((BLK * 5,), jnp.float32),
        pltpu.VMEM((BLK * 5,), jnp.float32),
        pltpu.VMEM((BLK,), jnp.float32),
        pltpu.VMEM((BLK,), jnp.float32),
        pltpu.VMEM((BLK,), jnp.float32),
        pltpu.VMEM((L,), jnp.float32),
        pltpu.VMEM_SHARED((NPAD,), jnp.float32),
        pltpu.SemaphoreType.DMA,
    ],
)(_pass_b)


def _pass_c(px, py, pz, nbr, tq_t, dsc_t, d_in, i_in, zeros4,
            g_out, gi_out,
            px_v, py_v, pz_v, nbr_v, tq_v, dsv, den_v, dtmp_v,
            jb_v, valj_v, gi_v, gacc):
    core = lax.axis_index("c")
    sub = lax.axis_index("s")
    wid = core * 16 + sub
    base = wid * BLK
    iota = lax.iota(jnp.int32, L)
    lane0 = iota == 0
    zvec = jnp.zeros((L,), jnp.float32)

    pltpu.sync_copy(px, px_v)
    pltpu.sync_copy(py, py_v)
    pltpu.sync_copy(pz, pz_v)

    @pl.when(sub == 0)
    def _():
        pltpu.sync_copy(zeros4, gacc)

    pltpu.sync_copy(d_in.at[pl.ds(base, BLK)], den_v)
    pltpu.sync_copy(d_in.at[pl.ds(NPAD + base, BLK)], dtmp_v)

    def add_body(m, _):
        av = m * L + iota
        s = plsc.load_gather(den_v, [av]) + plsc.load_gather(dtmp_v, [av])
        plsc.store_scatter(den_v, [av], s)
        return 0
    lax.fori_loop(0, BLK // L, add_body, 0)
    pltpu.sync_copy(i_in.at[pl.ds(wid * BLK, BLK)], dtmp_v)
    lax.fori_loop(0, BLK // L, add_body, 0)

    def zb(m, _):
        plsc.store_scatter(gi_v, [m * L + iota], zvec)
        return 0
    lax.fori_loop(0, BLK * 4 // L, zb, 0)

    def z4(m, _):
        plsc.store_scatter(valj_v, [m * L + iota, _splat_i(3)], zvec)
        return 0
    lax.fori_loop(0, G * NBR // L, z4, 0)
    plsc.subcore_barrier()

    def chunk_body(c, _):
        row0 = base + c * G
        pltpu.sync_copy(nbr.at[pl.ds(row0, G)], nbr_v)
        pltpu.sync_copy(tq_t.at[pl.ds(row0, G)], tq_v)
        pltpu.sync_copy(dsc_t.at[pl.ds(row0, G)], dsv)

        def ab(a, _):
            al = c * G + a
            agv = _splat_i(base + al)
            alv = _splat_i(al)
            xi = plsc.load_gather(px_v, [agv])
            yi = plsc.load_gather(py_v, [agv])
            zi3 = plsc.load_gather(pz_v, [agv])
            dei = plsc.load_gather(den_v, [alv])
            sx = jnp.float32(0.0)
            sy = jnp.float32(0.0)
            sz = jnp.float32(0.0)
            for h in range(2):
                nb = nbr_v[a, pl.ds(h * L, L)]
                valid = (nb < N) & (nb != agv)
                jj = jnp.where(valid, nb, 0)
                pos = a * NBR + h * L + iota
                dx = plsc.load_gather(px_v, [jj]) - xi
                dy = plsc.load_gather(py_v, [jj]) - yi
                dz = plsc.load_gather(pz_v, [jj]) - zi3
                gs = dsv[a, pl.ds(h * L, L)] - dei * tq_v[a, pl.ds(h * L, L)]
                gx = gs * dx
                gy = gs * dy
                gz = gs * dz
                plsc.store_scatter(jb_v, [pos], jj)
                plsc.store_scatter(valj_v, [pos, _splat_i(0)], gx)
                plsc.store_scatter(valj_v, [pos, _splat_i(1)], gy)
                plsc.store_scatter(valj_v, [pos, _splat_i(2)], gz)
                sx = sx + jnp.sum(gx)
                sy = sy + jnp.sum(gy)
                sz = sz + jnp.sum(gz)
            plsc.store_scatter(gi_v, [alv * 4 + 0], _splat_f(-sx), mask=lane0)
            plsc.store_scatter(gi_v, [alv * 4 + 1], _splat_f(-sy), mask=lane0)
            plsc.store_scatter(gi_v, [alv * 4 + 2], _splat_f(-sz), mask=lane0)
            return 0
        lax.fori_loop(0, G, ab, 0)
        pltpu.sync_copy(valj_v, gacc.at[jb_v], add=True)
        return 0

    lax.fori_loop(0, CH, chunk_body, 0)
    plsc.subcore_barrier()

    @pl.when(sub == 0)
    def _():
        pltpu.sync_copy(gacc, g_out.at[pl.ds(core * NPAD, NPAD)])
    pltpu.sync_copy(gi_v, gi_out.at[pl.ds(wid * BLK * 4, BLK * 4)])


_pass_c_call = functools.partial(
    pl.kernel,
    out_type=(jax.ShapeDtypeStruct((2 * NPAD, 4), jnp.float32),
              jax.ShapeDtypeStruct((NW * BLK * 4,), jnp.float32)),
    mesh=_MESH,
    compiler_params=_SC_PARAMS,
    scratch_types=[
        pltpu.VMEM((NPAD,), jnp.float32),
        pltpu.VMEM((NPAD,), jnp.float32),
        pltpu.VMEM((NPAD,), jnp.float32),
        pltpu.VMEM((G, NBR), jnp.int32),
        pltpu.VMEM((G, NBR), jnp.float32),
        pltpu.VMEM((G, NBR), jnp.float32),
        pltpu.VMEM((BLK,), jnp.float32),
        pltpu.VMEM((BLK,), jnp.float32),
        pltpu.VMEM((G * NBR,), jnp.int32),
        pltpu.VMEM((G * NBR, 4), jnp.float32),
        pltpu.VMEM((BLK * 4,), jnp.float32),
        pltpu.VMEM_SHARED((NPAD, 4), jnp.float32),
    ],
)(_pass_c)


def kernel(positions, numbers, neighbor_matrix, covalent_radii, r4r2, c6_reference, coord_num_ref):
    n = positions.shape[0]
    p = positions.astype(jnp.float32) * ANGSTROM_TO_BOHR
    px = jnp.zeros((NPAD,), jnp.float32).at[:n].set(p[:, 0])
    py = jnp.zeros((NPAD,), jnp.float32).at[:n].set(p[:, 1])
    pz = jnp.zeros((NPAD,), jnp.float32).at[:n].set(p[:, 2])
    z_t = jnp.zeros((NPAD,), jnp.int32).at[:n].set(numbers)
    cov_t = jnp.zeros((96,), jnp.float32).at[:95].set(covalent_radii)
    nbr = jnp.full((NPAD, NBR), 2**30, jnp.int32).at[:n].set(neighbor_matrix)

    cn_p, tq_p = _pass_a_call(px, py, pz, z_t, cov_t, nbr)
    cn = cn_p[:n]

    r4_t = jnp.zeros((96,), jnp.float32).at[:95].set(r4r2)
    cnr_t = jnp.zeros((480,), jnp.float32).at[:475].set(coord_num_ref.reshape(-1))
    c6_2d = jnp.zeros((95 * 95, MROW), jnp.float32).at[:, :25].set(c6_reference.reshape(95 * 95, 25))
    zeros1 = jnp.zeros((NPAD,), jnp.float32)
    zeros4 = jnp.zeros((NPAD, 4), jnp.float32)

    d_p, i_p, e_p, ds_p = _pass_b_call(
        px, py, pz, z_t, cn_p, r4_t, cnr_t, c6_2d, nbr, zeros1)
    g_p, gi_p = _pass_c_call(px, py, pz, nbr, tq_p, ds_p, d_p, i_p, zeros4)

    grad4 = g_p[:NPAD] + g_p[NPAD:] + gi_p.reshape(NPAD, 4)
    forces = -grad4[:n, :3] * (HARTREE_TO_EV / BOHR_TO_ANGSTROM)
    energy = jnp.sum(e_p) * HARTREE_TO_EV
    return (jnp.reshape(energy, (1,)), forces, cn)


# speedup vs baseline: 59.8768x; 1.5919x over previous
import functools

import jax
import jax.numpy as jnp
from jax import lax
from jax.experimental import pallas as pl
from jax.experimental.pallas import tpu as pltpu
from jax.experimental.pallas import tpu_sc as plsc

BOHR_TO_ANGSTROM = 0.529177210544
ANGSTROM_TO_BOHR = 1.0 / BOHR_TO_ANGSTROM
HARTREE_TO_EV = 27.211386245981
A1 = 0.3981
A2 = 4.4211
S8 = 1.9889
S6 = 1.0
K1 = 16.0
K3 = -4.0

N = 10000
NBR = 32
NPAD = 10240
NW = 32
BLK = NPAD // NW
G = 8
CH = BLK // G
PC = G * NBR
MROW = 32
L = 16

_MESH = plsc.VectorSubcoreMesh(core_axis_name="c", subcore_axis_name="s")
_SC_PARAMS = pltpu.CompilerParams(needs_layout_passes=False,
                                  use_tc_tiling_on_sc=False)


def _rsqrt_nr(x):
    i = plsc.bitcast(x, jnp.int32)
    i = jnp.int32(0x5F3759DF) - (i >> 1)
    y = plsc.bitcast(i, jnp.float32)
    for _ in range(3):
        y = y * (1.5 - 0.5 * x * y * y)
    return y


def _splat_f(x):
    return jnp.broadcast_to(x, (L,)).astype(jnp.float32)


def _splat_i(x):
    return jnp.broadcast_to(x, (L,)).astype(jnp.int32)


def _dedup_combine(jj, vals, iota):
    comb = list(vals)
    is_dup = iota < 0
    for s in range(1, L):
        idxs = (iota - s) & (L - 1)
        jr = jnp.take_along_axis(jj, idxs, axis=0)
        same = jj == jr
        for t in range(len(vals)):
            comb[t] = comb[t] + jnp.where(
                same, jnp.take_along_axis(vals[t], idxs, axis=0), 0.0)
        is_dup = is_dup | (same & (iota >= s))
    return comb, jnp.logical_not(is_dup)


def _pass_a(px, py, pz, z_t, cov_t, nbr, cn_out, tq_out,
            px_v, py_v, pz_v, z_v, cov_v, nbr_v, tq_v, cn_v):
    core = lax.axis_index("c")
    sub = lax.axis_index("s")
    wid = core * 16 + sub
    base = wid * BLK
    iota = lax.iota(jnp.int32, L)
    lane0 = iota == 0

    pltpu.sync_copy(px, px_v)
    pltpu.sync_copy(py, py_v)
    pltpu.sync_copy(pz, pz_v)
    pltpu.sync_copy(z_t, z_v)
    pltpu.sync_copy(cov_t, cov_v)
    pltpu.sync_copy(nbr.at[pl.ds(base, BLK)], nbr_v)

    def atom_body(a, _):
        agv = _splat_i(base + a)
        ziv = plsc.load_gather(z_v, [agv])
        covi = plsc.load_gather(cov_v, [ziv])
        xi = plsc.load_gather(px_v, [agv])
        yi = plsc.load_gather(py_v, [agv])
        zi = plsc.load_gather(pz_v, [agv])
        cn_a = jnp.float32(0.0)
        for h in range(2):
            nb = nbr_v[a, pl.ds(h * L, L)]
            valid = (nb < N) & (nb != agv)
            jj = jnp.where(valid, nb, 0)
            vm = jnp.where(valid, 1.0, 0.0).astype(jnp.float32)
            dx = plsc.load_gather(px_v, [jj]) - xi
            dy = plsc.load_gather(py_v, [jj]) - yi
            dz = plsc.load_gather(pz_v, [jj]) - zi
            d2 = dx * dx + dy * dy + dz * dz
            d2m = jnp.where(valid, d2, 1.0)
            y = _rsqrt_nr(d2m)
            zjv = plsc.load_gather(z_v, [jj])
            covj = plsc.load_gather(cov_v, [zjv])
            rc = covi + covj
            ee = jnp.exp(-K1 * (rc * y - 1.0))
            sig = 1.0 / (1.0 + ee)
            cn_a = cn_a + jnp.sum(vm * sig)
            tq = vm * sig * (1.0 - sig) * K1 * rc * (y * y * y)
            tq_v[a, pl.ds(h * L, L)] = tq
        plsc.store_scatter(cn_v, [_splat_i(a)], _splat_f(cn_a), mask=lane0)
        return 0
    lax.fori_loop(0, BLK, atom_body, 0)
    pltpu.sync_copy(tq_v, tq_out.at[pl.ds(base, BLK)])
    pltpu.sync_copy(cn_v, cn_out.at[pl.ds(base, BLK)])


_pass_a_call = functools.partial(
    pl.kernel,
    out_type=(jax.ShapeDtypeStruct((NPAD,), jnp.float32),
              jax.ShapeDtypeStruct((NPAD, NBR), jnp.float32)),
    mesh=_MESH,
    compiler_params=_SC_PARAMS,
    scratch_types=[
        pltpu.VMEM((NPAD,), jnp.float32),
        pltpu.VMEM((NPAD,), jnp.float32),
        pltpu.VMEM((NPAD,), jnp.float32),
        pltpu.VMEM((NPAD,), jnp.int32),
        pltpu.VMEM((96,), jnp.float32),
        pltpu.VMEM((BLK, NBR), jnp.int32),
        pltpu.VMEM((BLK, NBR), jnp.float32),
        pltpu.VMEM((BLK,), jnp.float32),
    ],
)(_pass_a)


def _pass_b(px, py, pz, z_t, cn_t, r4_t, cnr_t, c6_2d, nbr,
            d_out, e_out, ds_out,
            px_v, py_v, pz_v, z_v, cn_v, r4_v, cnr_v, nbr_v, m0, m1,
            midx_v, ds_v, w_v, u_v, ws_v, us_v, e_v, dacc_v, sem):
    core = lax.axis_index("c")
    sub = lax.axis_index("s")
    wid = core * 16 + sub
    base = wid * BLK
    iota = lax.iota(jnp.int32, L)
    lane0 = iota == 0
    zvec = jnp.zeros((L,), jnp.float32)

    pltpu.sync_copy(px, px_v)
    pltpu.sync_copy(py, py_v)
    pltpu.sync_copy(pz, pz_v)
    pltpu.sync_copy(z_t, z_v)
    pltpu.sync_copy(cn_t, cn_v)
    pltpu.sync_copy(r4_t, r4_v)
    pltpu.sync_copy(cnr_t, cnr_v)
    pltpu.sync_copy(nbr.at[pl.ds(base, BLK)], nbr_v)

    def zero_body(k, _):
        plsc.store_scatter(dacc_v, [k * L + iota], zvec)
        return 0
    lax.fori_loop(0, NPAD // L, zero_body, 0)

    def wu_body(k, _):
        lidx = k * L + iota
        al = lidx // 5
        q = lidx - al * 5
        ag = base + al
        cnv = plsc.load_gather(cn_v, [ag])
        zv = plsc.load_gather(z_v, [ag])
        cnr = plsc.load_gather(cnr_v, [zv * 5 + q])
        dcv = cnv - cnr
        wv = jnp.exp(K3 * dcv * dcv)
        plsc.store_scatter(w_v, [lidx], wv)
        plsc.store_scatter(u_v, [lidx], 2.0 * K3 * dcv * wv)
        return 0
    lax.fori_loop(0, BLK * 5 // L, wu_body, 0)

    def ws_body(m, _):
        av = m * L + iota
        aw = zvec
        au = zvec
        for q in range(5):
            aw = aw + plsc.load_gather(w_v, [av * 5 + q])
            au = au + plsc.load_gather(u_v, [av * 5 + q])
        plsc.store_scatter(ws_v, [av], aw)
        plsc.store_scatter(us_v, [av], au)
        return 0
    lax.fori_loop(0, BLK // L, ws_body, 0)

    def ib(k, _):
        a = k // 2
        hl = (k - a * 2) * L
        agv = _splat_i(base + a)
        nb = plsc.load_gather(nbr_v, [_splat_i(a), _splat_i(hl) + iota])
        valid = (nb < N) & (nb != agv)
        jj = jnp.where(valid, nb, 0)
        ziv = plsc.load_gather(z_v, [agv])
        zjv = plsc.load_gather(z_v, [jj])
        plsc.store_scatter(midx_v, [k * L + iota], ziv * 95 + zjv)
        return 0
    lax.fori_loop(0, BLK * NBR // L, ib, 0)

    pltpu.async_copy(c6_2d.at[midx_v.at[pl.ds(0, PC)]], m0, sem)

    def super_body(s, e_acc):
        for half, mb, mo in ((0, m0, m1), (1, m1, m0)):
            c = s * 2 + half
            pltpu.make_async_copy(
                c6_2d.at[midx_v.at[pl.ds(c * PC, PC)]], mb, sem).wait()
            nxt = c + 1

            @pl.when(nxt < CH)
            def _():
                start = (nxt - CH * (nxt // CH)) * PC
                pltpu.async_copy(
                    c6_2d.at[midx_v.at[pl.ds(start, PC)]], mo, sem)

            def ab(a, e_acc):
                al = c * G + a
                agv = _splat_i(base + al)
                xi = plsc.load_gather(px_v, [agv])
                yi = plsc.load_gather(py_v, [agv])
                zi3 = plsc.load_gather(pz_v, [agv])
                ziv = plsc.load_gather(z_v, [agv])
                r4i = plsc.load_gather(r4_v, [ziv])
                alv = _splat_i(al)
                wi = [plsc.load_gather(w_v, [alv * 5 + q]) for q in range(5)]
                ui = [plsc.load_gather(u_v, [alv * 5 + q]) for q in range(5)]
                wsi = plsc.load_gather(ws_v, [alv])
                usi = plsc.load_gather(us_v, [alv])
                s_i = jnp.float32(0.0)
                for h in range(2):
                    hl = h * L
                    nb = plsc.load_gather(nbr_v, [alv, _splat_i(hl) + iota])
                    valid = (nb < N) & (nb != agv)
                    jj = jnp.where(valid, nb, 0)
                    vm = jnp.where(valid, 1.0, 0.0).astype(jnp.float32)
                    mrow = _splat_i(a * NBR + hl) + iota
                    zjv = plsc.load_gather(z_v, [jj])
                    dx = plsc.load_gather(px_v, [jj]) - xi
                    dy = plsc.load_gather(py_v, [jj]) - yi
                    dz = plsc.load_gather(pz_v, [jj]) - zi3
                    d2 = dx * dx + dy * dy + dz * dz
                    d2m = jnp.where(valid, d2, 1.0)
                    cnj = plsc.load_gather(cn_v, [jj])
                    r4j = plsc.load_gather(r4_v, [zjv])
                    wj = []
                    uj = []
                    for rj in range(5):
                        cnr = plsc.load_gather(cnr_v, [zjv * 5 + rj])
                        dcj = cnj - cnr
                        wjr = jnp.exp(K3 * dcj * dcj)
                        wj.append(wjr)
                        uj.append(2.0 * K3 * dcj * wjr)
                    wsj = wj[0] + wj[1] + wj[2] + wj[3] + wj[4]
                    usj = uj[0] + uj[1] + uj[2] + uj[3] + uj[4]
                    num = zvec
                    dni = zvec
                    dnj = zvec
                    for q in range(5):
                        for rj in range(5):
                            mv = plsc.load_gather(
                                mb, [mrow, _splat_i(q * 5 + rj)])
                            t1 = mv * wi[q]
                            num = num + t1 * wj[rj]
                            dni = dni + (mv * ui[q]) * wj[rj]
                            dnj = dnj + t1 * uj[rj]
                    den_raw = wsi * wsj
                    den = jnp.maximum(den_raw, 1e-30)
                    m30 = den_raw >= 1e-30
                    c6 = num / den
                    dc6i = (dni - jnp.where(m30, c6 * usi * wsj, 0.0)) / den
                    dc6j = (dnj - jnp.where(m30, c6 * wsi * usj, 0.0)) / den
                    qq = 3.0 * r4i * r4j
                    yq = _rsqrt_nr(qq)
                    r0 = qq * yq
                    fd = A1 * r0 + A2
                    fd2 = fd * fd
                    f6 = fd2 * fd2 * fd2
                    f8 = f6 * fd2
                    r2 = d2m
                    r4p = r2 * r2
                    r6 = r4p * r2
                    r8 = r4p * r4p
                    inv6 = 1.0 / (r6 + f6)
                    inv8 = 1.0 / (r8 + f8)
                    gfact = S6 * inv6 + S8 * qq * inv8
                    af = -0.5 * vm * gfact
                    e_acc = e_acc + af * c6
                    jsc = jnp.where(valid, jj, N + iota)
                    (cval,), first = _dedup_combine(jsc, [af * dc6j], iota)
                    plsc.addupdate_scatter(dacc_v, [jsc], cval, mask=first)
                    ds_v[al, pl.ds(hl, L)] = vm * 0.5 * c6 * (
                        6.0 * S6 * r4p * inv6 * inv6
                        + 8.0 * S8 * qq * r6 * inv8 * inv8)
                    s_i = s_i + jnp.sum(af * dc6i)
                plsc.addupdate_scatter(dacc_v, [agv], _splat_f(s_i),
                                       mask=lane0)
                return e_acc
            e_acc = lax.fori_loop(0, G, ab, e_acc)
        return e_acc

    e_acc = lax.fori_loop(0, CH // 2, super_body,
                          jnp.zeros((L,), jnp.float32))
    pltpu.sync_copy(ds_v, ds_out.at[pl.ds(base, BLK)])
    pltpu.sync_copy(dacc_v, d_out.at[pl.ds(wid * NPAD, NPAD)])
    e_v[...] = e_acc
    pltpu.sync_copy(e_v, e_out.at[pl.ds(wid * L, L)])


_pass_b_call = functools.partial(
    pl.kernel,
    out_type=(jax.ShapeDtypeStruct((NW * NPAD,), jnp.float32),
              jax.ShapeDtypeStruct((NW * L,), jnp.float32),
              jax.ShapeDtypeStruct((NPAD, NBR), jnp.float32)),
    mesh=_MESH,
    compiler_params=_SC_PARAMS,
    scratch_types=[
        pltpu.VMEM((NPAD,), jnp.float32),
        pltpu.VMEM((NPAD,), jnp.float32),
        pltpu.VMEM((NPAD,), jnp.float32),
        pltpu.VMEM((NPAD,), jnp.int32),
        pltpu.VMEM((NPAD,), jnp.float32),
        pltpu.VMEM((96,), jnp.float32),
        pltpu.VMEM((480,), jnp.float32),
        pltpu.VMEM((BLK, NBR), jnp.int32),
        pltpu.VMEM((PC, MROW), jnp.float32),
        pltpu.VMEM((PC, MROW), jnp.float32),
        pltpu.VMEM((BLK * NBR,), jnp.int32),
        pltpu.VMEM((BLK, NBR), jnp.float32),
        pltpu.VMEM((BLK * 5,), jnp.float32),
        pltpu.VMEM((BLK * 5,), jnp.float32),
        pltpu.VMEM((BLK,), jnp.float32),
        pltpu.VMEM((BLK,), jnp.float32),
        pltpu.VMEM((L,), jnp.float32),
        pltpu.VMEM((NPAD,), jnp.float32),
        pltpu.SemaphoreType.DMA,
    ],
)(_pass_b)


def _pass_c(px, py, pz, nbr, tq_t, dsc_t, den_t,
            gx_out, gy_out, gz_out,
            px_v, py_v, pz_v, nbr_v, tq_v, dsv, den_v, gx_v, gy_v, gz_v):
    core = lax.axis_index("c")
    sub = lax.axis_index("s")
    wid = core * 16 + sub
    base = wid * BLK
    iota = lax.iota(jnp.int32, L)
    lane0 = iota == 0
    zvec = jnp.zeros((L,), jnp.float32)

    pltpu.sync_copy(px, px_v)
    pltpu.sync_copy(py, py_v)
    pltpu.sync_copy(pz, pz_v)
    pltpu.sync_copy(nbr.at[pl.ds(base, BLK)], nbr_v)
    pltpu.sync_copy(tq_t.at[pl.ds(base, BLK)], tq_v)
    pltpu.sync_copy(dsc_t.at[pl.ds(base, BLK)], dsv)
    pltpu.sync_copy(den_t.at[pl.ds(base, BLK)], den_v)

    def zero_body(k, _):
        pos = k * L + iota
        plsc.store_scatter(gx_v, [pos], zvec)
        plsc.store_scatter(gy_v, [pos], zvec)
        plsc.store_scatter(gz_v, [pos], zvec)
        return 0
    lax.fori_loop(0, NPAD // L, zero_body, 0)

    def ab(al, _):
        agv = _splat_i(base + al)
        alv = _splat_i(al)
        xi = plsc.load_gather(px_v, [agv])
        yi = plsc.load_gather(py_v, [agv])
        zi3 = plsc.load_gather(pz_v, [agv])
        dei = plsc.load_gather(den_v, [alv])
        sx = jnp.float32(0.0)
        sy = jnp.float32(0.0)
        sz = jnp.float32(0.0)
        for h in range(2):
            nb = nbr_v[al, pl.ds(h * L, L)]
            valid = (nb < N) & (nb != agv)
            jj = jnp.where(valid, nb, 0)
            dx = plsc.load_gather(px_v, [jj]) - xi
            dy = plsc.load_gather(py_v, [jj]) - yi
            dz = plsc.load_gather(pz_v, [jj]) - zi3
            gs = dsv[al, pl.ds(h * L, L)] - dei * tq_v[al, pl.ds(h * L, L)]
            gx = gs * dx
            gy = gs * dy
            gz = gs * dz
            jsc = jnp.where(valid, jj, N + iota)
            (cx, cy, cz), first = _dedup_combine(jsc, [gx, gy, gz], iota)
            plsc.addupdate_scatter(gx_v, [jsc], cx, mask=first)
            plsc.addupdate_scatter(gy_v, [jsc], cy, mask=first)
            plsc.addupdate_scatter(gz_v, [jsc], cz, mask=first)
            sx = sx + jnp.sum(gx)
            sy = sy + jnp.sum(gy)
            sz = sz + jnp.sum(gz)
        plsc.addupdate_scatter(gx_v, [agv], _splat_f(-sx), mask=lane0)
        plsc.addupdate_scatter(gy_v, [agv], _splat_f(-sy), mask=lane0)
        plsc.addupdate_scatter(gz_v, [agv], _splat_f(-sz), mask=lane0)
        return 0
    lax.fori_loop(0, BLK, ab, 0)
    pltpu.sync_copy(gx_v, gx_out.at[pl.ds(wid * NPAD, NPAD)])
    pltpu.sync_copy(gy_v, gy_out.at[pl.ds(wid * NPAD, NPAD)])
    pltpu.sync_copy(gz_v, gz_out.at[pl.ds(wid * NPAD, NPAD)])


_pass_c_call = functools.partial(
    pl.kernel,
    out_type=(jax.ShapeDtypeStruct((NW * NPAD,), jnp.float32),
              jax.ShapeDtypeStruct((NW * NPAD,), jnp.float32),
              jax.ShapeDtypeStruct((NW * NPAD,), jnp.float32)),
    mesh=_MESH,
    compiler_params=_SC_PARAMS,
    scratch_types=[
        pltpu.VMEM((NPAD,), jnp.float32),
        pltpu.VMEM((NPAD,), jnp.float32),
        pltpu.VMEM((NPAD,), jnp.float32),
        pltpu.VMEM((BLK, NBR), jnp.int32),
        pltpu.VMEM((BLK, NBR), jnp.float32),
        pltpu.VMEM((BLK, NBR), jnp.float32),
        pltpu.VMEM((BLK,), jnp.float32),
        pltpu.VMEM((NPAD,), jnp.float32),
        pltpu.VMEM((NPAD,), jnp.float32),
        pltpu.VMEM((NPAD,), jnp.float32),
    ],
)(_pass_c)


def kernel(positions, numbers, neighbor_matrix, covalent_radii, r4r2, c6_reference, coord_num_ref):
    n = positions.shape[0]
    p = positions.astype(jnp.float32) * ANGSTROM_TO_BOHR
    px = jnp.zeros((NPAD,), jnp.float32).at[:n].set(p[:, 0])
    py = jnp.zeros((NPAD,), jnp.float32).at[:n].set(p[:, 1])
    pz = jnp.zeros((NPAD,), jnp.float32).at[:n].set(p[:, 2])
    z_t = jnp.zeros((NPAD,), jnp.int32).at[:n].set(numbers)
    cov_t = jnp.zeros((96,), jnp.float32).at[:95].set(covalent_radii)
    nbr = jnp.full((NPAD, NBR), 2**30, jnp.int32).at[:n].set(neighbor_matrix)

    cn_p, tq_p = _pass_a_call(px, py, pz, z_t, cov_t, nbr)
    cn = cn_p[:n]

    r4_t = jnp.zeros((96,), jnp.float32).at[:95].set(r4r2)
    cnr_t = jnp.zeros((480,), jnp.float32).at[:475].set(coord_num_ref.reshape(-1))
    c6_2d = jnp.zeros((95 * 95, MROW), jnp.float32).at[:, :25].set(c6_reference.reshape(95 * 95, 25))

    d_p, e_p, ds_p = _pass_b_call(
        px, py, pz, z_t, cn_p, r4_t, cnr_t, c6_2d, nbr)
    dEdcn = jnp.sum(d_p.reshape(NW, NPAD), axis=0)
    gx_p, gy_p, gz_p = _pass_c_call(px, py, pz, nbr, tq_p, ds_p, dEdcn)

    grad = jnp.stack([jnp.sum(gx_p.reshape(NW, NPAD), axis=0)[:n],
                      jnp.sum(gy_p.reshape(NW, NPAD), axis=0)[:n],
                      jnp.sum(gz_p.reshape(NW, NPAD), axis=0)[:n]], axis=1)
    forces = -grad * (HARTREE_TO_EV / BOHR_TO_ANGSTROM)
    energy = jnp.sum(e_p) * HARTREE_TO_EV
    return (jnp.reshape(energy, (1,)), forces, cn)
